# baseline (speedup 1.0000x reference)
import jax
import jax.numpy as jnp
from jax import lax
from jax.experimental import pallas as pl
from jax.experimental.pallas import tpu as pltpu

_N = 1024
_HID = 64
_M = 2
_SEM_HID = 128
_BI = 256
_NB = _N // _BI
_SLOPE = 0.2


def _fused_kernel(adj_hbm, feat_ref, aw_ref, gat_W_ref, gat_al_ref,
                  gat_ar_ref, gat_b_ref, gm_W_ref, gm_al_ref, gm_ar_ref,
                  gm_b_ref, sem_W1_ref, sem_b1_ref, sem_q_ref, ft_W_ref,
                  ft_b_ref, out_ref, abuf, h_s, el_s, elb_s, erowb_s, ecol_s,
                  rhs1_s, rhs2_s, acc1_s, acc2_s, sem):
    f32 = jnp.float32
    j = pl.program_id(0)

    def copy(blk):
        return pltpu.make_async_copy(
            adj_hbm.at[:, pl.ds(blk * _BI, _BI), :],
            abuf.at[blk % 2], sem.at[blk % 2])

    @pl.when(j == 0)
    def _():
        copy(0).start()

    @pl.when(j + 1 < _NB)
    def _():
        copy(j + 1).start()

    @pl.when(j == 0)
    def _():
        feat = feat_ref[...]
        params = ((gat_W_ref[...], gat_al_ref[...].reshape(1, _HID),
                   gat_ar_ref[...].reshape(1, _HID)),
                  (gm_W_ref[0], gm_al_ref[0:1, :], gm_ar_ref[0:1, :]),
                  (gm_W_ref[1], gm_al_ref[1:2, :], gm_ar_ref[1:2, :]))
        for g, (W, al, ar) in enumerate(params):
            h = jnp.dot(feat, W, preferred_element_type=f32)
            el = jnp.sum(h * al, axis=1, keepdims=True)
            elmax = jnp.max(el)
            u1 = jnp.exp(el - elmax)
            u2 = jnp.exp(_SLOPE * (el - elmax))
            h_s[g] = h
            el_s[g] = el
            elb_s[g] = el.astype(jnp.bfloat16)
            erow = lax.dot_general(ar, h, (((1,), (1,)), ((), ())),
                                   preferred_element_type=f32)
            erowb_s[g] = erow.astype(jnp.bfloat16)
            ecol_s[g] = jnp.sum(h * ar, axis=1, keepdims=True)
            rhs1_s[g] = jnp.concatenate([h * u1, u1],
                                        axis=1).astype(jnp.bfloat16)
            rhs2_s[g] = jnp.concatenate([h * u2, u2],
                                        axis=1).astype(jnp.bfloat16)

    copy(j).wait()

    bf = jnp.bfloat16
    a0b = abuf[j % 2, 0, :, :].astype(bf)
    a1b = abuf[j % 2, 1, :, :].astype(bf)
    w = jax.nn.softmax(aw_ref[...].reshape(1, _M))
    a0e = jnp.where(w[0, 0] != 0.0, a0b, bf(0.0))
    a1e = jnp.where(w[0, 1] != 0.0, a1b, bf(0.0))
    cnt_m = jnp.maximum(a0e, a1e)

    dn = (((0,), (0,)), ((), ()))
    for g, cnt in ((0, cnt_m), (1, a0b), (2, a1b)):
        el_blk = elb_s[g, pl.ds(j * _BI, _BI), :]
        x = el_blk + erowb_s[g]
        m1 = jnp.where(x >= bf(0.0), cnt, bf(0.0))
        m2 = cnt - m1
        rhs1 = rhs1_s[g, pl.ds(j * _BI, _BI), :]
        rhs2 = rhs2_s[g, pl.ds(j * _BI, _BI), :]
        r1 = lax.dot_general(m1, rhs1, dn, preferred_element_type=f32)
        r2 = lax.dot_general(m2, rhs2, dn, preferred_element_type=f32)

        @pl.when(j == 0)
        def _():
            acc1_s[g] = r1
            acc2_s[g] = r2

        @pl.when(j > 0)
        def _():
            acc1_s[g] += r1
            acc2_s[g] += r2

    @pl.when(j == _NB - 1)
    def _():
        zs = []
        for g in range(3):
            h = h_s[g]
            el = el_s[g]
            elmax = jnp.max(el)
            er_col = ecol_s[g]
            t = elmax + er_col
            c = jnp.where(t >= 0.0, t, _SLOPE * t)
            f1 = jnp.exp(t - c)
            f2 = jnp.exp(_SLOPE * t - c)
            xd = el + er_col
            ed = jnp.where(xd >= 0.0, xd, _SLOPE * xd)
            term = jnp.exp(ed - c)
            A1 = acc1_s[g]
            A2 = acc2_s[g]
            num = f1 * A1[:, :_HID] + f2 * A2[:, :_HID] + term * h
            den = (f1 * A1[:, _HID:_HID + 1] + f2 * A2[:, _HID:_HID + 1]
                   + term)
            zs.append(num / den)
        mg = jnp.tanh(zs[0] + gat_b_ref[...].reshape(1, _HID))
        m0 = jnp.tanh(zs[1] + gm_b_ref[0:1, :])
        m1_ = jnp.tanh(zs[2] + gm_b_ref[1:2, :])

        sem_W1 = sem_W1_ref[...]
        sem_b1 = sem_b1_ref[...].reshape(1, _SEM_HID)
        sem_q = sem_q_ref[...].reshape(1, _SEM_HID)

        def wp(xv):
            tt = jnp.tanh(jnp.dot(xv, sem_W1, preferred_element_type=f32)
                          + sem_b1)
            return jnp.sum(tt * sem_q)

        s0 = wp(mg) / _N
        s1 = wp(m0) / _N
        s2 = wp(m1_) / _N
        smax = jnp.maximum(jnp.maximum(s0, s1), s2)
        e0 = jnp.exp(s0 - smax)
        e1 = jnp.exp(s1 - smax)
        e2 = jnp.exp(s2 - smax)
        tot = e0 + e1 + e2
        semantic = (e0 / tot) * mg + (e1 / tot) * m0 + (e2 / tot) * m1_

        ft_W = ft_W_ref[...]
        fa = (jnp.dot(mg, ft_W[0:_HID, :], preferred_element_type=f32)
              + jnp.dot(semantic, ft_W[_HID:2 * _HID, :],
                        preferred_element_type=f32)
              + ft_b_ref[...].reshape(1, _HID))
        out_ref[...] = jnp.tanh(fa)


def kernel(adj_list, feat, attention_weights, gat_W, gat_al, gat_ar, gat_b,
           gm_W, gm_al, gm_ar, gm_b, sem_W1, sem_b1, sem_q, ft_W, ft_b):
    full = lambda shape: pl.BlockSpec(shape, lambda j: (0,) * len(shape))
    out = pl.pallas_call(
        _fused_kernel,
        grid=(_NB,),
        in_specs=[
            pl.BlockSpec(memory_space=pl.ANY),
            full((_N, _HID)),
            full((_M,)),
            full((_HID, _HID)),
            full((_HID,)),
            full((_HID,)),
            full((_HID,)),
            full((_M, _HID, _HID)),
            full((_M, _HID)),
            full((_M, _HID)),
            full((_M, _HID)),
            full((_HID, _SEM_HID)),
            full((_SEM_HID,)),
            full((_SEM_HID,)),
            full((2 * _HID, _HID)),
            full((_HID,)),
        ],
        out_specs=pl.BlockSpec((_N, _HID), lambda j: (0, 0)),
        out_shape=jax.ShapeDtypeStruct((_N, _HID), jnp.float32),
        scratch_shapes=[
            pltpu.VMEM((2, _M, _BI, _N), jnp.int32),
            pltpu.VMEM((3, _N, _HID), jnp.float32),
            pltpu.VMEM((3, _N, 1), jnp.float32),
            pltpu.VMEM((3, _N, 1), jnp.bfloat16),
            pltpu.VMEM((3, 1, _N), jnp.bfloat16),
            pltpu.VMEM((3, _N, 1), jnp.float32),
            pltpu.VMEM((3, _N, _HID + 1), jnp.bfloat16),
            pltpu.VMEM((3, _N, _HID + 1), jnp.bfloat16),
            pltpu.VMEM((3, _N, _HID + 1), jnp.float32),
            pltpu.VMEM((3, _N, _HID + 1), jnp.float32),
            pltpu.SemaphoreType.DMA((2,)),
        ],
    )(adj_list, feat, attention_weights, gat_W, gat_al, gat_ar, gat_b,
      gm_W, gm_al, gm_ar, gm_b, sem_W1, sem_b1, sem_q, ft_W, ft_b)
    return out
